# baseline (speedup 1.0000x reference)
import functools

import jax
import jax.numpy as jnp
from jax import lax
from jax.experimental import pallas as pl
from jax.experimental.pallas import tpu as pltpu, tpu_sc as plsc

D_MODEL = 1024
BATCH = 4
SEQ = 4096

_info = plsc.get_sparse_core_info()
NC, NS, LANES = _info.num_cores, _info.num_subcores, _info.num_lanes
NW = NC * NS

CHUNK = 8
CW = CHUNK * D_MODEL
SEQ_PER_W = SEQ // NW
N_CHUNKS = SEQ_PER_W // CHUNK
N_TASKS = N_CHUNKS * BATCH
RING = 8
UNROLL = 8


def _body(x_hbm, pos_hbm, out_hbm, xb, posb, in_sems, out_sems, pos_sems):
    cid = lax.axis_index("c")
    sid = lax.axis_index("s")
    wid = sid * NC + cid
    seq0 = wid * SEQ_PER_W

    def x_off(t):
        ci, b = divmod(t, BATCH)
        return b * (SEQ * D_MODEL) + (seq0 + ci * CHUNK) * D_MODEL

    def start_in(t):
        slot = t % RING
        return pltpu.async_copy(
            x_hbm.at[pl.ds(x_off(t), CW)], xb.at[slot], in_sems.at[slot])

    def start_out(t):
        slot = t % RING
        return pltpu.async_copy(
            xb.at[slot], out_hbm.at[pl.ds(x_off(t), CW)], out_sems.at[slot])

    def start_pos(ci):
        p = ci % 2
        return pltpu.async_copy(
            pos_hbm.at[pl.ds((seq0 + ci * CHUNK) * D_MODEL, CW)],
            posb.at[p], pos_sems.at[p])

    pos_d = {0: start_pos(0)}
    in_d = {t: start_in(t) for t in range(RING)}
    out_d = {}

    for t in range(N_TASKS):
        ci, b = divmod(t, BATCH)
        p = ci % 2
        slot = t % RING
        j = t - 1 + RING
        if t >= 1 and j < N_TASKS:
            out_d[t - 1].wait()
            in_d[j] = start_in(j)
        if b == 0:
            if ci + 1 < N_CHUNKS:
                pos_d[ci + 1] = start_pos(ci + 1)
            pos_d[ci].wait()
        in_d[t].wait()

        def _add(i, _, _p=p, _slot=slot):
            base = i * (LANES * UNROLL)
            for u in range(UNROLL):
                off = base + u * LANES
                v = posb[_p, pl.ds(off, LANES)]
                plsc.addupdate(xb.at[_slot, pl.ds(off, LANES)], v)
            return 0

        lax.fori_loop(0, CW // (LANES * UNROLL), _add, 0)

        out_d[t] = start_out(t)

    for t in range(N_TASKS - RING, N_TASKS):
        out_d[t].wait()


@jax.jit
def kernel(x, pos_table):
    mesh = plsc.VectorSubcoreMesh(core_axis_name="c", subcore_axis_name="s")
    out_flat = pl.kernel(
        _body,
        out_type=jax.ShapeDtypeStruct((BATCH * SEQ * D_MODEL,), jnp.float32),
        mesh=mesh,
        scratch_types=[
            pltpu.VMEM((RING, CW), jnp.float32),
            pltpu.VMEM((2, CW), jnp.float32),
            pltpu.SemaphoreType.DMA((RING,)),
            pltpu.SemaphoreType.DMA((RING,)),
            pltpu.SemaphoreType.DMA((2,)),
        ],
    )(x.reshape(-1), pos_table.reshape(-1))
    return out_flat.reshape(BATCH, SEQ, D_MODEL)


# speedup vs baseline: 3.6516x; 3.6516x over previous
import jax
import jax.numpy as jnp
from jax import lax
from jax.experimental import pallas as pl
from jax.experimental.pallas import tpu as pltpu, tpu_sc as plsc

D_MODEL = 1024
BATCH = 4
SEQ = 4096

_info = plsc.get_sparse_core_info()
NC, NS, LANES = _info.num_cores, _info.num_subcores, _info.num_lanes
NW = NC * NS

CHUNK = 8
CW = CHUNK * D_MODEL
SEQ_PER_W = SEQ // NW
N_CHUNKS = SEQ_PER_W // CHUNK
N_TASKS = N_CHUNKS * BATCH
RING = 8
GROUP = 16
N_GROUPS = CW // (GROUP * LANES)


def _group_slices(i):
    r = i >> 2
    coff = (i & 3) << 8
    return [(r, coff + u * LANES) for u in range(GROUP)]


def _body(x_hbm, pos_hbm, out_hbm, xb, posb, in_sems, out_sems, pos_sems):
    cid = lax.axis_index("c")
    sid = lax.axis_index("s")
    wid = sid * NC + cid
    seq0 = wid * SEQ_PER_W

    def start_in(t):
        ci, b = divmod(t, BATCH)
        slot = t % RING
        return pltpu.async_copy(
            x_hbm.at[b, pl.ds(seq0 + ci * CHUNK, CHUNK), :],
            xb.at[slot], in_sems.at[slot])

    def start_out(t):
        ci, b = divmod(t, BATCH)
        slot = t % RING
        return pltpu.async_copy(
            xb.at[slot], out_hbm.at[b, pl.ds(seq0 + ci * CHUNK, CHUNK), :],
            out_sems.at[slot])

    def start_pos(ci):
        p = ci % 2
        return pltpu.async_copy(
            pos_hbm.at[pl.ds(seq0 + ci * CHUNK, CHUNK), :],
            posb.at[p], pos_sems.at[p])

    pos_d = {0: start_pos(0)}
    in_d = {t: start_in(t) for t in range(RING)}
    out_d = {}

    for t in range(N_TASKS):
        ci, b = divmod(t, BATCH)
        p = ci % 2
        slot = t % RING
        j = t - 1 + RING
        if t >= 1 and j < N_TASKS:
            out_d[t - 1].wait()
            in_d[j] = start_in(j)
        if b == 0:
            if ci + 1 < N_CHUNKS:
                pos_d[ci + 1] = start_pos(ci + 1)
            pos_d[ci].wait()
        in_d[t].wait()

        def _load(i):
            r = i >> 2
            coff = pl.multiple_of((i & 3) << 8, 256)
            return tuple(
                posb[p, r, pl.ds(coff + u * LANES, LANES)]
                for u in range(GROUP))

        def _step(i, vs, _slot=slot):
            r = i >> 2
            coff = pl.multiple_of((i & 3) << 8, 256)
            i1 = (i + 1) & (N_GROUPS - 1)
            nxt = _load(i1)
            for u in range(GROUP):
                plsc.addupdate(
                    xb.at[_slot, r, pl.ds(coff + u * LANES, LANES)], vs[u])
            return nxt

        lax.fori_loop(0, N_GROUPS, _step, _load(0))

        out_d[t] = start_out(t)

    for t in range(N_TASKS - RING, N_TASKS):
        out_d[t].wait()


@jax.jit
def kernel(x, pos_table):
    mesh = plsc.VectorSubcoreMesh(core_axis_name="c", subcore_axis_name="s")
    return pl.kernel(
        _body,
        out_type=jax.ShapeDtypeStruct((BATCH, SEQ, D_MODEL), jnp.float32),
        mesh=mesh,
        scratch_types=[
            pltpu.VMEM((RING, CHUNK, D_MODEL), jnp.float32),
            pltpu.VMEM((2, CHUNK, D_MODEL), jnp.float32),
            pltpu.SemaphoreType.DMA((RING,)),
            pltpu.SemaphoreType.DMA((RING,)),
            pltpu.SemaphoreType.DMA((2,)),
        ],
        compiler_params=pltpu.CompilerParams(use_tc_tiling_on_sc=True),
    )(x, pos_table)
